# Initial kernel scaffold; baseline (speedup 1.0000x reference)
import jax
import jax.numpy as jnp
from jax.experimental import pallas as pl


def kernel(pos_in_feat, pos_edge_index, neg_in_feat, neg_edge_index, weight1, weight2, bias1, bias2, prelu_alpha, bil_w1, bil_b1, bil_w2, bil_b2):
    raise NotImplementedError("write your pallas kernel here")


# speedup vs baseline: 2.9928x; 2.9928x over previous
import functools

import jax
import jax.numpy as jnp
from jax import lax
from jax.experimental import pallas as pl
from jax.experimental.pallas import tpu as pltpu
from jax.experimental.pallas import tpu_sc as plsc

_N = 10000
_E = 320000
_DIN = 128
_DOUT = 64
_S = 4
_B = _N // _S
_BP = 2560
_NR = 4 * _BP

_NC = 2
_NS = 16
_NW = _NC * _NS
_EW = _E // _NW
_CH = 80
_NCHUNK = _EW // _CH

_mesh = plsc.VectorSubcoreMesh(core_axis_name="c", subcore_axis_name="s")
_sc_params = pltpu.CompilerParams(use_tc_tiling_on_sc=False)


@functools.partial(
    pl.kernel,
    out_type=(
        jax.ShapeDtypeStruct((2, _NC, _NR, 16), jnp.float32),
        jax.ShapeDtypeStruct((2, _NC, _NR, 16), jnp.float32),
    ),
    mesh=_mesh,
    scratch_types=[
        pltpu.VMEM_SHARED((_NR, 16), jnp.float32),
        pltpu.VMEM_SHARED((_NR, 16), jnp.float32),
        pltpu.VMEM_SHARED((_NR, 16), jnp.float32),
        pltpu.VMEM_SHARED((_NR, 16), jnp.float32),
        pltpu.VMEM((_CH,), jnp.int32),
        pltpu.VMEM((_CH,), jnp.int32),
        pltpu.VMEM((_CH,), jnp.int32),
        pltpu.VMEM((_CH, 16), jnp.float32),
    ],
    compiler_params=_sc_params,
)
def _s1(src_p, dst_p, src_n, dst_n, z16, o16,
        cs_out, cd_out,
        cs_pos, cd_pos, cs_neg, cd_neg, sbuf, dbuf, tbuf, ones_v):
    cid = lax.axis_index("c")
    sid = lax.axis_index("s")
    wid = sid * _NC + cid
    rr = _NR // _NS
    pltpu.sync_copy(o16, ones_v)
    pltpu.sync_copy(z16.at[pl.ds(sid * rr, rr)], cs_pos.at[pl.ds(sid * rr, rr)])
    pltpu.sync_copy(z16.at[pl.ds(sid * rr, rr)], cs_neg.at[pl.ds(sid * rr, rr)])
    pltpu.sync_copy(z16.at[pl.ds(sid * rr, rr)], cd_pos.at[pl.ds(sid * rr, rr)])
    pltpu.sync_copy(z16.at[pl.ds(sid * rr, rr)], cd_neg.at[pl.ds(sid * rr, rr)])
    plsc.subcore_barrier()

    for src_h, dst_h, cs, cd in ((src_p, dst_p, cs_pos, cd_pos),
                                 (src_n, dst_n, cs_neg, cd_neg)):
        def chunk(c, _, src_h=src_h, dst_h=dst_h, cs=cs, cd=cd):
            off = wid * _EW + c * _CH
            pltpu.sync_copy(src_h.at[pl.ds(off, _CH)], sbuf)
            pltpu.sync_copy(dst_h.at[pl.ds(off, _CH)], dbuf)
            for i in range(_CH // 16):
                d = dbuf[pl.ds(i * 16, 16)]
                tbuf[pl.ds(i * 16, 16)] = (d & 3) * _BP + (d >> 2)
            pltpu.sync_copy(ones_v, cs.at[sbuf], add=True)
            pltpu.sync_copy(ones_v, cd.at[tbuf], add=True)
            return 0
        lax.fori_loop(0, _NCHUNK, chunk, 0)

    plsc.subcore_barrier()
    pltpu.sync_copy(cs_pos.at[pl.ds(sid * rr, rr)],
                    cs_out.at[0, cid, pl.ds(sid * rr, rr)])
    pltpu.sync_copy(cs_neg.at[pl.ds(sid * rr, rr)],
                    cs_out.at[1, cid, pl.ds(sid * rr, rr)])
    pltpu.sync_copy(cd_pos.at[pl.ds(sid * rr, rr)],
                    cd_out.at[0, cid, pl.ds(sid * rr, rr)])
    pltpu.sync_copy(cd_neg.at[pl.ds(sid * rr, rr)],
                    cd_out.at[1, cid, pl.ds(sid * rr, rr)])


_T2R = 2000


def _t2_body(feat_ref, w_ref, cnt_ref, out_ref):
    cnt = cnt_ref[0, :, 0:1] + cnt_ref[1, :, 0:1]
    scale = lax.rsqrt(jnp.maximum(cnt, 1.0))
    r = lax.broadcasted_iota(jnp.int32, (_T2R, 1), 0)
    scale = jnp.where((r % _S) == 0, 0.0, scale)
    x = feat_ref[...] * scale
    out_ref[...] = jnp.dot(x, w_ref[0], preferred_element_type=jnp.float32)


def _t2(feat, wstack, cnt):
    return pl.pallas_call(
        _t2_body,
        grid=(2, _N // _T2R),
        in_specs=[
            pl.BlockSpec((_T2R, _DIN), lambda j, i: (i, 0)),
            pl.BlockSpec((1, _DIN, _DOUT), lambda j, i: (j, 0, 0)),
            pl.BlockSpec((_NC, _T2R, 16), lambda j, i: (0, i, 0)),
        ],
        out_specs=pl.BlockSpec((_T2R, _DOUT), lambda j, i: (j * (_N // _T2R) + i, 0)),
        out_shape=jax.ShapeDtypeStruct((2 * _N, _DOUT), jnp.float32),
    )(feat, wstack, cnt)


@functools.partial(
    pl.kernel,
    out_type=(
        jax.ShapeDtypeStruct((_NC, _NR, _DOUT), jnp.float32),
        jax.ShapeDtypeStruct((_NC, _NR, _DOUT), jnp.float32),
    ),
    mesh=_mesh,
    scratch_types=[
        pltpu.VMEM_SHARED((_NR, _DOUT), jnp.float32),
        pltpu.VMEM_SHARED((_NR, _DOUT), jnp.float32),
        pltpu.VMEM((_CH,), jnp.int32),
        pltpu.VMEM((_CH,), jnp.int32),
        pltpu.VMEM((_CH,), jnp.int32),
        pltpu.VMEM((_CH,), jnp.int32),
        pltpu.VMEM((_CH, _DOUT), jnp.float32),
        pltpu.SemaphoreType.DMA,
    ],
    compiler_params=_sc_params,
)
def _s2(p_pos, p_neg, src_p, dst_p, src_n, dst_n, z64,
        agg_pos, agg_neg,
        acc_pos, acc_neg, sbuf, dbuf, gbuf, tbuf, rows, sem):
    cid = lax.axis_index("c")
    sid = lax.axis_index("s")
    wid = sid * _NC + cid
    rr = _NR // _NS
    pltpu.sync_copy(z64.at[pl.ds(sid * rr, rr)], acc_pos.at[pl.ds(sid * rr, rr)])
    pltpu.sync_copy(z64.at[pl.ds(sid * rr, rr)], acc_neg.at[pl.ds(sid * rr, rr)])
    plsc.subcore_barrier()

    for p_h, src_h, dst_h, acc in ((p_pos, src_p, dst_p, acc_pos),
                                   (p_neg, src_n, dst_n, acc_neg)):
        def chunk(c, _, p_h=p_h, src_h=src_h, dst_h=dst_h, acc=acc):
            off = wid * _EW + c * _CH
            pltpu.sync_copy(src_h.at[pl.ds(off, _CH)], sbuf)
            pltpu.sync_copy(dst_h.at[pl.ds(off, _CH)], dbuf)
            for i in range(_CH // 16):
                s = sbuf[pl.ds(i * 16, 16)]
                d = dbuf[pl.ds(i * 16, 16)]
                gbuf[pl.ds(i * 16, 16)] = s + jnp.where((d & 3) == 0, _N, 0)
                tbuf[pl.ds(i * 16, 16)] = (d & 3) * _BP + (d >> 2)
            pltpu.async_copy(p_h.at[gbuf], rows, sem).wait()
            pltpu.sync_copy(rows, acc.at[tbuf], add=True)
            return 0
        lax.fori_loop(0, _NCHUNK, chunk, 0)

    plsc.subcore_barrier()
    pltpu.sync_copy(acc_pos.at[pl.ds(sid * rr, rr)],
                    agg_pos.at[cid, pl.ds(sid * rr, rr)])
    pltpu.sync_copy(acc_neg.at[pl.ds(sid * rr, rr)],
                    agg_neg.at[cid, pl.ds(sid * rr, rr)])


_T3G = 512


def _prelu(x, a):
    return jnp.where(x >= 0, x, a * x)


def _rownorm(x):
    return x * lax.rsqrt(jnp.maximum(jnp.sum(x * x, axis=1, keepdims=True),
                                     1e-24))


def _t3_body(rec_ref, pool_ref, rcnt_ref, pcnt_ref, anch_ref,
             w_ref, b_ref, alpha_ref, bw_ref, bb_ref,
             rdt_ref, rsc_ref):
    alpha = alpha_ref[0, 0]
    b1 = b_ref[0:1, :]
    b2 = b_ref[1:2, :]

    rec = rec_ref[0, 0] + rec_ref[0, 1]
    rc = rcnt_ref[0, 0, :, 0:1] + rcnt_ref[0, 1, :, 0:1]
    rh = _prelu(rec * lax.rsqrt(jnp.maximum(rc, 1.0)) + b2, alpha)
    rn = _rownorm(rh)

    pool = jnp.zeros((_T3G, _DOUT), jnp.float32)
    for k in range(3):
        pk = pool_ref[0, k] + pool_ref[0, 3 + k]
        ck = pcnt_ref[0, k, :, 0:1] + pcnt_ref[0, 3 + k, :, 0:1]
        pool = pool + _prelu(pk * lax.rsqrt(jnp.maximum(ck, 1.0)) + b1, alpha)
    pn = _rownorm(pool / 3.0)

    a = anch_ref[0]
    a1 = _rownorm(_prelu(jnp.dot(a, w_ref[0], preferred_element_type=jnp.float32) + b1, alpha))
    a2 = _rownorm(_prelu(jnp.dot(a, w_ref[1], preferred_element_type=jnp.float32) + b2, alpha))

    rdt_ref[0] = (jnp.sum(jnp.dot(pn, bw_ref[0], preferred_element_type=jnp.float32) * a1,
                          axis=1, keepdims=True) + bb_ref[0, 0])
    rsc_ref[0] = (jnp.sum(jnp.dot(rn, bw_ref[1], preferred_element_type=jnp.float32) * a2,
                          axis=1, keepdims=True) + bb_ref[1, 0])


def _t3(rec_acc, pool_acc, rcnt, pcnt, anchors, wstack, bstack, alpha, bws, bbs):
    nblk = _BP // _T3G
    return pl.pallas_call(
        _t3_body,
        grid=(2, nblk),
        in_specs=[
            pl.BlockSpec((1, _NC, _T3G, _DOUT), lambda b, i: (b, 0, i, 0)),
            pl.BlockSpec((1, 2 * 3, _T3G, _DOUT), lambda b, i: (b, 0, i, 0)),
            pl.BlockSpec((1, _NC, _T3G, 16), lambda b, i: (b, 0, i, 0)),
            pl.BlockSpec((1, 2 * 3, _T3G, 16), lambda b, i: (b, 0, i, 0)),
            pl.BlockSpec((1, _T3G, _DIN), lambda b, i: (b, i, 0)),
            pl.BlockSpec((2, _DIN, _DOUT), lambda b, i: (0, 0, 0)),
            pl.BlockSpec((2, _DOUT), lambda b, i: (0, 0)),
            pl.BlockSpec((1, 1), lambda b, i: (0, 0)),
            pl.BlockSpec((2, _DOUT, _DOUT), lambda b, i: (0, 0, 0)),
            pl.BlockSpec((2, 1), lambda b, i: (0, 0)),
        ],
        out_specs=[
            pl.BlockSpec((1, _T3G, 1), lambda b, i: (b, i, 0)),
            pl.BlockSpec((1, _T3G, 1), lambda b, i: (b, i, 0)),
        ],
        out_shape=[
            jax.ShapeDtypeStruct((2, _BP, 1), jnp.float32),
            jax.ShapeDtypeStruct((2, _BP, 1), jnp.float32),
        ],
    )(rec_acc, pool_acc, rcnt, pcnt, anchors, wstack, bstack, alpha, bws, bbs)


def kernel(pos_in_feat, pos_edge_index, neg_in_feat, neg_edge_index,
           weight1, weight2, bias1, bias2, prelu_alpha,
           bil_w1, bil_b1, bil_w2, bil_b2):
    src_p, dst_p = pos_edge_index[0], pos_edge_index[1]
    src_n, dst_n = neg_edge_index[0], neg_edge_index[1]

    z16 = jnp.zeros((_NR, 16), jnp.float32)
    o16 = jnp.ones((_CH, 16), jnp.float32)
    z64 = jnp.zeros((_NR, _DOUT), jnp.float32)

    cs, cd = _s1(src_p, dst_p, src_n, dst_n, z16, o16)

    wstack = jnp.stack([weight1, weight2])
    p_pos = _t2(pos_in_feat, wstack, cs[0])
    p_neg = _t2(neg_in_feat, wstack, cs[1])

    agg_pos, agg_neg = _s2(p_pos, p_neg, src_p, dst_p, src_n, dst_n, z64)

    agg = jnp.stack([agg_pos, agg_neg]).reshape(2, _NC, 4, _BP, _DOUT)
    rec_acc = agg[:, :, 0]
    pool_acc = agg[:, :, 1:4].reshape(2, _NC * 3, _BP, _DOUT)
    cdr = cd.reshape(2, _NC, 4, _BP, 16)
    rcnt = cdr[:, :, 0]
    pcnt = cdr[:, :, 1:4].reshape(2, _NC * 3, _BP, 16)

    anch = jnp.stack([pos_in_feat, neg_in_feat]).reshape(2, _B, _S, _DIN)[:, :, 0, :]
    anch = jnp.pad(anch, ((0, 0), (0, _BP - _B), (0, 0)))

    bstack = jnp.stack([bias1, bias2])
    alpha = prelu_alpha.reshape(1, 1).astype(jnp.float32)
    bws = jnp.concatenate([bil_w1, bil_w2], axis=0)
    bbs = jnp.stack([bil_b1, bil_b2])

    rdt, rsc = _t3(rec_acc, pool_acc, rcnt, pcnt, anch,
                   wstack, bstack, alpha, bws, bbs)

    return (rdt[0, :_B], rsc[0, :_B], rdt[1, :_B], rsc[1, :_B])


# speedup vs baseline: 7.3319x; 2.4498x over previous
import functools

import jax
import jax.numpy as jnp
from jax import lax
from jax.experimental import pallas as pl
from jax.experimental.pallas import tpu as pltpu
from jax.experimental.pallas import tpu_sc as plsc

_N = 10000
_E = 320000
_DIN = 128
_DOUT = 64
_S = 4
_B = _N // _S
_BP = 2560
_NR = 4 * _BP

_NC = 2
_NS = 16
_NW = _NC * _NS
_EW = _E // _NW
_CH = 80
_NCHUNK = _EW // _CH

_mesh = plsc.VectorSubcoreMesh(core_axis_name="c", subcore_axis_name="s")
_sc_params = pltpu.CompilerParams(use_tc_tiling_on_sc=False)


_K = 5
_NG = _NCHUNK // _K


@functools.partial(
    pl.kernel,
    out_type=(
        jax.ShapeDtypeStruct((2, _NC, _NR, 16), jnp.float32),
        jax.ShapeDtypeStruct((2, _NC, _NR, 16), jnp.float32),
    ),
    mesh=_mesh,
    scratch_types=[
        pltpu.VMEM_SHARED((_NR, 16), jnp.float32),
        pltpu.VMEM_SHARED((_NR, 16), jnp.float32),
        pltpu.VMEM_SHARED((_NR, 16), jnp.float32),
        pltpu.VMEM_SHARED((_NR, 16), jnp.float32),
        pltpu.VMEM((_NCHUNK, _CH), jnp.int32),
        pltpu.VMEM((_NCHUNK, _CH), jnp.int32),
        pltpu.VMEM((_CH, 16), jnp.float32),
        pltpu.SemaphoreType.DMA,
    ],
    compiler_params=_sc_params,
)
def _s1(src_p, dst_p, src_n, dst_n, z16, o16,
        cs_out, cd_out,
        cs_pos, cd_pos, cs_neg, cd_neg, sB, dB, ones_v, sem):
    cid = lax.axis_index("c")
    sid = lax.axis_index("s")
    wid = sid * _NC + cid
    rr = _NR // _NS
    pltpu.sync_copy(o16, ones_v)
    pltpu.sync_copy(z16.at[pl.ds(sid * rr, rr)], cs_pos.at[pl.ds(sid * rr, rr)])
    pltpu.sync_copy(z16.at[pl.ds(sid * rr, rr)], cs_neg.at[pl.ds(sid * rr, rr)])
    pltpu.sync_copy(z16.at[pl.ds(sid * rr, rr)], cd_pos.at[pl.ds(sid * rr, rr)])
    pltpu.sync_copy(z16.at[pl.ds(sid * rr, rr)], cd_neg.at[pl.ds(sid * rr, rr)])
    plsc.subcore_barrier()

    for src_h, dst_h, cs, cd in ((src_p, dst_p, cs_pos, cd_pos),
                                 (src_n, dst_n, cs_neg, cd_neg)):
        pltpu.sync_copy(src_h.at[wid], sB)
        pltpu.sync_copy(dst_h.at[wid], dB)

        def comp(c, _):
            for i in range(_CH // 16):
                sl = pl.ds(i * 16, 16)
                d = dB[c, sl]
                dB[c, sl] = (d & 3) * _BP + (d >> 2)
            return 0
        lax.fori_loop(0, _NCHUNK, comp, 0)

        def scat(g, _, cs=cs, cd=cd):
            for r in range(_K):
                c = g * _K + r
                pltpu.async_copy(ones_v, cs.at[sB.at[c]], sem, add=True)
                pltpu.async_copy(ones_v, cd.at[dB.at[c]], sem, add=True)
            for r in range(_K):
                c = g * _K + r
                pltpu.make_async_copy(ones_v, cs.at[sB.at[c]], sem).wait()
                pltpu.make_async_copy(ones_v, cd.at[dB.at[c]], sem).wait()
            return 0
        lax.fori_loop(0, _NG, scat, 0)

    plsc.subcore_barrier()
    pltpu.sync_copy(cs_pos.at[pl.ds(sid * rr, rr)],
                    cs_out.at[0, cid, pl.ds(sid * rr, rr)])
    pltpu.sync_copy(cs_neg.at[pl.ds(sid * rr, rr)],
                    cs_out.at[1, cid, pl.ds(sid * rr, rr)])
    pltpu.sync_copy(cd_pos.at[pl.ds(sid * rr, rr)],
                    cd_out.at[0, cid, pl.ds(sid * rr, rr)])
    pltpu.sync_copy(cd_neg.at[pl.ds(sid * rr, rr)],
                    cd_out.at[1, cid, pl.ds(sid * rr, rr)])


_T2R = 2000


def _t2_body(feat_ref, w_ref, cnt_ref, out_ref):
    cnt = cnt_ref[0, :, 0:1] + cnt_ref[1, :, 0:1]
    scale = lax.rsqrt(jnp.maximum(cnt, 1.0))
    r = lax.broadcasted_iota(jnp.int32, (_T2R, 1), 0)
    scale = jnp.where((r % _S) == 0, 0.0, scale)
    x = feat_ref[...] * scale
    out_ref[...] = jnp.dot(x, w_ref[0], preferred_element_type=jnp.float32)


def _t2(feat, wstack, cnt):
    return pl.pallas_call(
        _t2_body,
        grid=(2, _N // _T2R),
        in_specs=[
            pl.BlockSpec((_T2R, _DIN), lambda j, i: (i, 0)),
            pl.BlockSpec((1, _DIN, _DOUT), lambda j, i: (j, 0, 0)),
            pl.BlockSpec((_NC, _T2R, 16), lambda j, i: (0, i, 0)),
        ],
        out_specs=pl.BlockSpec((_T2R, _DOUT), lambda j, i: (j * (_N // _T2R) + i, 0)),
        out_shape=jax.ShapeDtypeStruct((2 * _N, _DOUT), jnp.float32),
    )(feat, wstack, cnt)


@functools.partial(
    pl.kernel,
    out_type=(
        jax.ShapeDtypeStruct((_NC, _NR, _DOUT), jnp.float32),
        jax.ShapeDtypeStruct((_NC, _NR, _DOUT), jnp.float32),
    ),
    mesh=_mesh,
    scratch_types=[
        pltpu.VMEM_SHARED((_NR, _DOUT), jnp.float32),
        pltpu.VMEM_SHARED((_NR, _DOUT), jnp.float32),
        pltpu.VMEM((_NCHUNK, _CH), jnp.int32),
        pltpu.VMEM((_NCHUNK, _CH), jnp.int32),
        pltpu.VMEM((_K * _CH, _DOUT), jnp.float32),
        pltpu.SemaphoreType.DMA,
        pltpu.SemaphoreType.DMA,
    ],
    compiler_params=_sc_params,
)
def _s2(p_pos, p_neg, src_p, dst_p, src_n, dst_n, z64,
        agg_pos, agg_neg,
        acc_pos, acc_neg, gB, tB, rows, gsem, ssem):
    cid = lax.axis_index("c")
    sid = lax.axis_index("s")
    wid = sid * _NC + cid
    rr = _NR // _NS
    pltpu.sync_copy(z64.at[pl.ds(sid * rr, rr)], acc_pos.at[pl.ds(sid * rr, rr)])
    pltpu.sync_copy(z64.at[pl.ds(sid * rr, rr)], acc_neg.at[pl.ds(sid * rr, rr)])
    plsc.subcore_barrier()

    for p_h, src_h, dst_h, acc in ((p_pos, src_p, dst_p, acc_pos),
                                   (p_neg, src_n, dst_n, acc_neg)):
        pltpu.sync_copy(src_h.at[wid], gB)
        pltpu.sync_copy(dst_h.at[wid], tB)

        def comp(c, _):
            for i in range(_CH // 16):
                sl = pl.ds(i * 16, 16)
                s = gB[c, sl]
                d = tB[c, sl]
                gB[c, sl] = s + jnp.where((d & 3) == 0, _N, 0)
                tB[c, sl] = (d & 3) * _BP + (d >> 2)
            return 0
        lax.fori_loop(0, _NCHUNK, comp, 0)

        def grp(g, _, p_h=p_h, acc=acc):
            for r in range(_K):
                c = g * _K + r
                pltpu.async_copy(p_h.at[gB.at[c]],
                                 rows.at[pl.ds(r * _CH, _CH)], gsem)
            for r in range(_K):
                c = g * _K + r
                pltpu.make_async_copy(p_h.at[gB.at[c]],
                                      rows.at[pl.ds(r * _CH, _CH)], gsem).wait()
            for r in range(_K):
                c = g * _K + r
                pltpu.async_copy(rows.at[pl.ds(r * _CH, _CH)],
                                 acc.at[tB.at[c]], ssem, add=True)
            for r in range(_K):
                c = g * _K + r
                pltpu.make_async_copy(rows.at[pl.ds(r * _CH, _CH)],
                                      acc.at[tB.at[c]], ssem).wait()
            return 0
        lax.fori_loop(0, _NG, grp, 0)

    plsc.subcore_barrier()
    pltpu.sync_copy(acc_pos.at[pl.ds(sid * rr, rr)],
                    agg_pos.at[cid, pl.ds(sid * rr, rr)])
    pltpu.sync_copy(acc_neg.at[pl.ds(sid * rr, rr)],
                    agg_neg.at[cid, pl.ds(sid * rr, rr)])


_T3G = 512


def _prelu(x, a):
    return jnp.where(x >= 0, x, a * x)


def _rownorm(x):
    return x * lax.rsqrt(jnp.maximum(jnp.sum(x * x, axis=1, keepdims=True),
                                     1e-24))


def _t3_body(rec_ref, pool_ref, rcnt_ref, pcnt_ref, anch_ref,
             w_ref, b_ref, alpha_ref, bw_ref, bb_ref,
             rdt_ref, rsc_ref):
    alpha = alpha_ref[0, 0]
    b1 = b_ref[0:1, :]
    b2 = b_ref[1:2, :]

    rec = rec_ref[0, 0] + rec_ref[0, 1]
    rc = rcnt_ref[0, 0, :, 0:1] + rcnt_ref[0, 1, :, 0:1]
    rh = _prelu(rec * lax.rsqrt(jnp.maximum(rc, 1.0)) + b2, alpha)
    rn = _rownorm(rh)

    pool = jnp.zeros((_T3G, _DOUT), jnp.float32)
    for k in range(3):
        pk = pool_ref[0, k] + pool_ref[0, 3 + k]
        ck = pcnt_ref[0, k, :, 0:1] + pcnt_ref[0, 3 + k, :, 0:1]
        pool = pool + _prelu(pk * lax.rsqrt(jnp.maximum(ck, 1.0)) + b1, alpha)
    pn = _rownorm(pool / 3.0)

    a = anch_ref[0]
    a1 = _rownorm(_prelu(jnp.dot(a, w_ref[0], preferred_element_type=jnp.float32) + b1, alpha))
    a2 = _rownorm(_prelu(jnp.dot(a, w_ref[1], preferred_element_type=jnp.float32) + b2, alpha))

    rdt_ref[0] = (jnp.sum(jnp.dot(pn, bw_ref[0], preferred_element_type=jnp.float32) * a1,
                          axis=1, keepdims=True) + bb_ref[0, 0])
    rsc_ref[0] = (jnp.sum(jnp.dot(rn, bw_ref[1], preferred_element_type=jnp.float32) * a2,
                          axis=1, keepdims=True) + bb_ref[1, 0])


def _t3(rec_acc, pool_acc, rcnt, pcnt, anchors, wstack, bstack, alpha, bws, bbs):
    nblk = _BP // _T3G
    return pl.pallas_call(
        _t3_body,
        grid=(2, nblk),
        in_specs=[
            pl.BlockSpec((1, _NC, _T3G, _DOUT), lambda b, i: (b, 0, i, 0)),
            pl.BlockSpec((1, 2 * 3, _T3G, _DOUT), lambda b, i: (b, 0, i, 0)),
            pl.BlockSpec((1, _NC, _T3G, 16), lambda b, i: (b, 0, i, 0)),
            pl.BlockSpec((1, 2 * 3, _T3G, 16), lambda b, i: (b, 0, i, 0)),
            pl.BlockSpec((1, _T3G, _DIN), lambda b, i: (b, i, 0)),
            pl.BlockSpec((2, _DIN, _DOUT), lambda b, i: (0, 0, 0)),
            pl.BlockSpec((2, _DOUT), lambda b, i: (0, 0)),
            pl.BlockSpec((1, 1), lambda b, i: (0, 0)),
            pl.BlockSpec((2, _DOUT, _DOUT), lambda b, i: (0, 0, 0)),
            pl.BlockSpec((2, 1), lambda b, i: (0, 0)),
        ],
        out_specs=[
            pl.BlockSpec((1, _T3G, 1), lambda b, i: (b, i, 0)),
            pl.BlockSpec((1, _T3G, 1), lambda b, i: (b, i, 0)),
        ],
        out_shape=[
            jax.ShapeDtypeStruct((2, _BP, 1), jnp.float32),
            jax.ShapeDtypeStruct((2, _BP, 1), jnp.float32),
        ],
    )(rec_acc, pool_acc, rcnt, pcnt, anchors, wstack, bstack, alpha, bws, bbs)


def kernel(pos_in_feat, pos_edge_index, neg_in_feat, neg_edge_index,
           weight1, weight2, bias1, bias2, prelu_alpha,
           bil_w1, bil_b1, bil_w2, bil_b2):
    src_p = pos_edge_index[0].reshape(_NW, _NCHUNK, _CH)
    dst_p = pos_edge_index[1].reshape(_NW, _NCHUNK, _CH)
    src_n = neg_edge_index[0].reshape(_NW, _NCHUNK, _CH)
    dst_n = neg_edge_index[1].reshape(_NW, _NCHUNK, _CH)

    z16 = jnp.zeros((_NR, 16), jnp.float32)
    o16 = jnp.ones((_CH, 16), jnp.float32)
    z64 = jnp.zeros((_NR, _DOUT), jnp.float32)

    cs, cd = _s1(src_p, dst_p, src_n, dst_n, z16, o16)

    wstack = jnp.stack([weight1, weight2])
    p_pos = _t2(pos_in_feat, wstack, cs[0])
    p_neg = _t2(neg_in_feat, wstack, cs[1])

    agg_pos, agg_neg = _s2(p_pos, p_neg, src_p, dst_p, src_n, dst_n, z64)

    agg = jnp.stack([agg_pos, agg_neg]).reshape(2, _NC, 4, _BP, _DOUT)
    rec_acc = agg[:, :, 0]
    pool_acc = agg[:, :, 1:4].reshape(2, _NC * 3, _BP, _DOUT)
    cdr = cd.reshape(2, _NC, 4, _BP, 16)
    rcnt = cdr[:, :, 0]
    pcnt = cdr[:, :, 1:4].reshape(2, _NC * 3, _BP, 16)

    anch = jnp.stack([pos_in_feat, neg_in_feat]).reshape(2, _B, _S, _DIN)[:, :, 0, :]
    anch = jnp.pad(anch, ((0, 0), (0, _BP - _B), (0, 0)))

    bstack = jnp.stack([bias1, bias2])
    alpha = prelu_alpha.reshape(1, 1).astype(jnp.float32)
    bws = jnp.concatenate([bil_w1, bil_w2], axis=0)
    bbs = jnp.stack([bil_b1, bil_b2])

    rdt, rsc = _t3(rec_acc, pool_acc, rcnt, pcnt, anch,
                   wstack, bstack, alpha, bws, bbs)

    return (rdt[0, :_B], rsc[0, :_B], rdt[1, :_B], rsc[1, :_B])
